# baseline (speedup 1.0000x reference)
import jax
import jax.numpy as jnp
from jax.experimental import pallas as pl
from jax.experimental.pallas import tpu as pltpu

_INV_SQRT2 = 0.7071067811865476


def _gelu(x):
    return 0.5 * x * (1.0 + jax.lax.erf(x * _INV_SQRT2))


def _dot_t(x, w):
    return jax.lax.dot_general(
        x, w, dimension_numbers=(((1,), (1,)), ((), ())),
        preferred_element_type=jnp.float32)


def _make_body(Bb, V, H, I):
    def _body(idx_ref, vs_ref, vals_ref, W1_ref, b1_ref, W2_ref, b2_ref,
              G1_ref, g1_ref, G2_ref, g2_ref, out_ref):
        vs = vs_ref[...]
        idx = idx_ref[...]

        idx_f = idx.astype(jnp.float32)
        sup_cols = []
        for i in range(I):
            s = None
            for j in range(i + 1, I):
                c = (idx[:, i:i + 1] == idx[:, j:j + 1])
                s = c if s is None else jnp.logical_or(s, c)
            if s is None:
                sup_cols.append(jnp.zeros((Bb, 1), jnp.float32))
            else:
                sup_cols.append(jnp.where(s, 1.0, 0.0))
        combo = jnp.concatenate([idx_f] + sup_cols, axis=1)

        ii = jax.lax.broadcasted_iota(jnp.int32, (Bb, Bb), 0)
        jj = jax.lax.broadcasted_iota(jnp.int32, (Bb, Bb), 1)
        eye = jnp.where(ii == jj, 1.0, 0.0)
        comboT = jax.lax.dot_general(
            combo, eye, dimension_numbers=(((0,), (0,)), ((), ())),
            preferred_element_type=jnp.float32)

        lane_b = jax.lax.broadcasted_iota(jnp.int32, (1, Bb), 1)
        siota = jax.lax.broadcasted_iota(jnp.int32, (Bb * V, Bb), 0)
        chunks = []
        for i in range(I):
            tt = lane_b * V + comboT[i:i + 1, :].astype(jnp.int32)
            tt = jnp.where(comboT[I + i:I + i + 1, :] > 0.5, -1, tt)
            ttb = jax.lax.broadcast_in_dim(tt, (Bb * V, Bb), (0, 1))
            chunks.append(jnp.where(siota == ttb, 1.0, 0.0))
        S = jnp.concatenate(chunks, axis=1).astype(jnp.bfloat16)

        vs_bf = vs.astype(jnp.bfloat16)
        orig = jax.lax.dot_general(
            S, vs_bf, dimension_numbers=(((0,), (0,)), ((), ())),
            preferred_element_type=jnp.float32)
        vals_bf = jnp.concatenate(
            [vals_ref[:, i * H:(i + 1) * H] for i in range(I)], axis=0)
        vals = vals_bf.astype(jnp.float32)

        W1a = W1_ref[:, :H]
        W1b = W1_ref[:, H:]
        h = _gelu(_dot_t(orig.astype(jnp.bfloat16), W1a)
                  + _dot_t(vals_bf, W1b) + b1_ref[...])
        enc = _dot_t(h.astype(jnp.bfloat16), W2_ref[...]) + b2_ref[...]
        g = _gelu(_dot_t(enc.astype(jnp.bfloat16), G1_ref[...]) + g1_ref[...])
        gate = jax.nn.sigmoid(
            jnp.sum(g * G2_ref[...], axis=-1, keepdims=True) + g2_ref[0, 0])
        delta = (gate * (vals - orig)).astype(jnp.bfloat16)

        out_ref[...] = vs + jax.lax.dot_general(
            S, delta, dimension_numbers=(((1,), (0,)), ((), ())),
            preferred_element_type=jnp.float32)
    return _body


@jax.jit
def _run(variable_states, intervention_indices, intervention_values,
         W1, b1, W2, b2, G1, g1, G2, g2):
    B, V, H = variable_states.shape
    I = intervention_indices.shape[1]
    Bb = 64
    grid = (B // Bb,)
    vs2 = variable_states.reshape(B * V, H)
    vals2 = intervention_values.reshape(B, I * H).astype(jnp.bfloat16)
    W1c = W1.astype(jnp.bfloat16)
    W2c = W2.astype(jnp.bfloat16)
    G1c = G1.astype(jnp.bfloat16)
    b1r = b1.reshape(1, H)
    b2r = b2.reshape(1, H)
    g1r = g1.reshape(1, H)
    g2r = g2.reshape(1, 1)
    full = lambda *shape: pl.BlockSpec(shape, lambda b: (0,) * len(shape))
    out = pl.pallas_call(
        _make_body(Bb, V, H, I),
        grid=grid,
        in_specs=[
            pl.BlockSpec((Bb, I), lambda b: (b, 0)),
            pl.BlockSpec((Bb * V, H), lambda b: (b, 0)),
            pl.BlockSpec((Bb, I * H), lambda b: (b, 0)),
            full(H, 2 * H),
            full(1, H),
            full(H, H),
            full(1, H),
            full(H, H),
            full(1, H),
            full(1, H),
            full(1, 1),
        ],
        out_specs=pl.BlockSpec((Bb * V, H), lambda b: (b, 0)),
        out_shape=jax.ShapeDtypeStruct((B * V, H), jnp.float32),
        compiler_params=pltpu.CompilerParams(
            dimension_semantics=("parallel",)),
    )(intervention_indices, vs2, vals2,
      W1c, b1r, W2c, b2r, G1c, g1r, G2, g2r)
    return out.reshape(B, V, H)


def kernel(variable_states, edge_probs, intervention_indices,
           intervention_values, W1, b1, W2, b2, G1, g1, G2, g2):
    del edge_probs
    return _run(variable_states, intervention_indices, intervention_values,
                W1, b1, W2, b2, G1, g1, G2, g2)


# speedup vs baseline: 1.1574x; 1.1574x over previous
import jax
import jax.numpy as jnp
from jax.experimental import pallas as pl
from jax.experimental.pallas import tpu as pltpu

_INV_SQRT2 = 0.7071067811865476


def _gelu(x):
    return 0.5 * x * (1.0 + jax.lax.erf(x * _INV_SQRT2))


def _dot_t(x, w):
    return jax.lax.dot_general(
        x, w, dimension_numbers=(((1,), (1,)), ((), ())),
        preferred_element_type=jnp.float32)


def _make_body(Bb, V, H, I):
    def _vals_dma(vals_hbm, scr, sems, g, slot):
        return [pltpu.make_async_copy(
            vals_hbm.at[pl.ds(g * Bb, Bb), i],
            scr.at[slot, i], sems.at[slot, i]) for i in range(I)]

    def _body(idx_ref, vs_ref, vals_hbm, W1_ref, b1_ref, W2_ref, b2_ref,
              G1_ref, g1_ref, G2_ref, g2_ref, out_ref, scr, sems):
        g = pl.program_id(0)
        ng = pl.num_programs(0)
        slot = jax.lax.rem(g, 2)

        @pl.when(g == 0)
        def _():
            for c in _vals_dma(vals_hbm, scr, sems, g, slot):
                c.start()

        @pl.when(g + 1 < ng)
        def _():
            for c in _vals_dma(vals_hbm, scr, sems, g + 1,
                               jax.lax.rem(g + 1, 2)):
                c.start()

        vs = vs_ref[...]
        idx = idx_ref[...]

        idx_f = idx.astype(jnp.float32)
        sup_cols = []
        for i in range(I):
            s = None
            for j in range(i + 1, I):
                c = (idx[:, i:i + 1] == idx[:, j:j + 1])
                s = c if s is None else jnp.logical_or(s, c)
            if s is None:
                sup_cols.append(jnp.zeros((Bb, 1), jnp.float32))
            else:
                sup_cols.append(jnp.where(s, 1.0, 0.0))
        combo = jnp.concatenate([idx_f] + sup_cols, axis=1)

        ii = jax.lax.broadcasted_iota(jnp.int32, (Bb, Bb), 0)
        jj = jax.lax.broadcasted_iota(jnp.int32, (Bb, Bb), 1)
        eye = jnp.where(ii == jj, 1.0, 0.0)
        comboT = jax.lax.dot_general(
            combo, eye, dimension_numbers=(((0,), (0,)), ((), ())),
            preferred_element_type=jnp.float32)

        lane_b = jax.lax.broadcasted_iota(jnp.int32, (1, Bb), 1)
        siota = jax.lax.broadcasted_iota(jnp.int32, (Bb * V, Bb), 0)
        chunks = []
        for i in range(I):
            tt = lane_b * V + comboT[i:i + 1, :].astype(jnp.int32)
            tt = jnp.where(comboT[I + i:I + i + 1, :] > 0.5, -1, tt)
            ttb = jax.lax.broadcast_in_dim(tt, (Bb * V, Bb), (0, 1))
            chunks.append(jnp.where(siota == ttb, 1.0, 0.0))
        S = jnp.concatenate(chunks, axis=1).astype(jnp.bfloat16)

        vs_bf = vs.astype(jnp.bfloat16)
        orig = jax.lax.dot_general(
            S, vs_bf, dimension_numbers=(((0,), (0,)), ((), ())),
            preferred_element_type=jnp.float32)

        for c in _vals_dma(vals_hbm, scr, sems, g, slot):
            c.wait()
        vals = jnp.concatenate([scr[slot, i] for i in range(I)], axis=0)

        W1a = W1_ref[:, :H]
        W1b = W1_ref[:, H:]
        h = _gelu(_dot_t(orig.astype(jnp.bfloat16), W1a)
                  + _dot_t(vals.astype(jnp.bfloat16), W1b) + b1_ref[...])
        enc = _dot_t(h.astype(jnp.bfloat16), W2_ref[...]) + b2_ref[...]
        g_ = _gelu(_dot_t(enc.astype(jnp.bfloat16), G1_ref[...]) + g1_ref[...])
        gate = jax.nn.sigmoid(
            jnp.sum(g_ * G2_ref[...], axis=-1, keepdims=True) + g2_ref[0, 0])
        delta = (gate * (vals - orig)).astype(jnp.bfloat16)

        out_ref[...] = vs + jax.lax.dot_general(
            S, delta, dimension_numbers=(((1,), (0,)), ((), ())),
            preferred_element_type=jnp.float32)
    return _body


@jax.jit
def _run(variable_states, intervention_indices, intervention_values,
         W1, b1, W2, b2, G1, g1, G2, g2):
    B, V, H = variable_states.shape
    I = intervention_indices.shape[1]
    Bb = 64
    grid = (B // Bb,)
    vs2 = variable_states.reshape(B * V, H)
    W1c = W1.astype(jnp.bfloat16)
    W2c = W2.astype(jnp.bfloat16)
    G1c = G1.astype(jnp.bfloat16)
    b1r = b1.reshape(1, H)
    b2r = b2.reshape(1, H)
    g1r = g1.reshape(1, H)
    g2r = g2.reshape(1, 1)
    full = lambda *shape: pl.BlockSpec(shape, lambda b: (0,) * len(shape))
    out = pl.pallas_call(
        _make_body(Bb, V, H, I),
        grid=grid,
        in_specs=[
            pl.BlockSpec((Bb, I), lambda b: (b, 0)),
            pl.BlockSpec((Bb * V, H), lambda b: (b, 0)),
            pl.BlockSpec(memory_space=pltpu.MemorySpace.HBM),
            full(H, 2 * H),
            full(1, H),
            full(H, H),
            full(1, H),
            full(H, H),
            full(1, H),
            full(1, H),
            full(1, 1),
        ],
        out_specs=pl.BlockSpec((Bb * V, H), lambda b: (b, 0)),
        out_shape=jax.ShapeDtypeStruct((B * V, H), jnp.float32),
        scratch_shapes=[
            pltpu.VMEM((2, I, Bb, H), jnp.float32),
            pltpu.SemaphoreType.DMA((2, I)),
        ],
        compiler_params=pltpu.CompilerParams(
            dimension_semantics=("arbitrary",)),
    )(intervention_indices, vs2, intervention_values,
      W1c, b1r, W2c, b2r, G1c, g1r, G2, g2r)
    return out.reshape(B, V, H)


def kernel(variable_states, edge_probs, intervention_indices,
           intervention_values, W1, b1, W2, b2, G1, g1, G2, g2):
    del edge_probs
    return _run(variable_states, intervention_indices, intervention_values,
                W1, b1, W2, b2, G1, g1, G2, g2)


# speedup vs baseline: 1.2260x; 1.0592x over previous
import jax
import jax.numpy as jnp
from jax.experimental import pallas as pl
from jax.experimental.pallas import tpu as pltpu

_INV_SQRT2 = 0.7071067811865476


def _gelu(x):
    return 0.5 * x * (1.0 + jax.lax.erf(x * _INV_SQRT2))


def _dot_t(x, w):
    return jax.lax.dot_general(
        x, w, dimension_numbers=(((1,), (1,)), ((), ())),
        preferred_element_type=jnp.float32)


def _make_body(Bb, V, H, I):
    def _vals_dma(vals_hbm, scr, sems, g, slot):
        return [pltpu.make_async_copy(
            vals_hbm.at[pl.ds(g * Bb, Bb), i],
            scr.at[slot, i], sems.at[slot, i]) for i in range(I)]

    def _body(idx_ref, vs_ref, vals_hbm, W1_ref, b1_ref, W2_ref, b2_ref,
              G1_ref, g1_ref, G2_ref, g2_ref, out_ref, scr, sems):
        g = pl.program_id(0)
        ng = pl.num_programs(0)
        slot = jax.lax.rem(g, 2)

        @pl.when(g == 0)
        def _():
            for c in _vals_dma(vals_hbm, scr, sems, g, slot):
                c.start()

        @pl.when(g + 1 < ng)
        def _():
            for c in _vals_dma(vals_hbm, scr, sems, g + 1,
                               jax.lax.rem(g + 1, 2)):
                c.start()

        vs = vs_ref[...]
        idx = idx_ref[...]

        idx_f = idx.astype(jnp.float32)
        sup_cols = []
        for i in range(I):
            s = None
            for j in range(i + 1, I):
                c = (idx[:, i:i + 1] == idx[:, j:j + 1])
                s = c if s is None else jnp.logical_or(s, c)
            if s is None:
                sup_cols.append(jnp.zeros((Bb, 1), jnp.float32))
            else:
                sup_cols.append(jnp.where(s, 1.0, 0.0))
        combo = jnp.concatenate([idx_f] + sup_cols, axis=1)

        ii = jax.lax.broadcasted_iota(jnp.int32, (Bb, Bb), 0)
        jj = jax.lax.broadcasted_iota(jnp.int32, (Bb, Bb), 1)
        eye = jnp.where(ii == jj, 1.0, 0.0)
        comboT = jax.lax.dot_general(
            combo, eye, dimension_numbers=(((0,), (0,)), ((), ())),
            preferred_element_type=jnp.float32)

        lane_b = jax.lax.broadcasted_iota(jnp.int32, (1, Bb), 1)
        siota = jax.lax.broadcasted_iota(jnp.int32, (Bb * V, Bb), 0)
        chunks = []
        for i in range(I):
            tt = lane_b * V + comboT[i:i + 1, :].astype(jnp.int32)
            tt = jnp.where(comboT[I + i:I + i + 1, :] > 0.5, -1, tt)
            ttb = jax.lax.broadcast_in_dim(tt, (Bb * V, Bb), (0, 1))
            chunks.append(jnp.where(siota == ttb, 1.0, 0.0))
        S = jnp.concatenate(chunks, axis=1).astype(jnp.bfloat16)

        vs_bf = vs.astype(jnp.bfloat16)
        orig = jax.lax.dot_general(
            S, vs_bf, dimension_numbers=(((0,), (0,)), ((), ())),
            preferred_element_type=jnp.float32)

        for c in _vals_dma(vals_hbm, scr, sems, g, slot):
            c.wait()
        vals = jnp.concatenate([scr[slot, i] for i in range(I)], axis=0)

        W1a = W1_ref[:, :H]
        W1b = W1_ref[:, H:]
        h = _gelu(_dot_t(orig, W1a) + _dot_t(vals, W1b) + b1_ref[...])
        enc = _dot_t(h, W2_ref[...]) + b2_ref[...]
        g_ = _gelu(_dot_t(enc, G1_ref[...]) + g1_ref[...])
        gate = jax.nn.sigmoid(
            jnp.sum(g_ * G2_ref[...], axis=-1, keepdims=True) + g2_ref[0, 0])
        delta = (gate * (vals - orig)).astype(jnp.bfloat16)

        out_ref[...] = vs + jax.lax.dot_general(
            S, delta, dimension_numbers=(((1,), (0,)), ((), ())),
            preferred_element_type=jnp.float32)
    return _body


@jax.jit
def _run(variable_states, intervention_indices, intervention_values,
         W1, b1, W2, b2, G1, g1, G2, g2):
    B, V, H = variable_states.shape
    I = intervention_indices.shape[1]
    Bb = 64
    grid = (B // Bb,)
    vs2 = variable_states.reshape(B * V, H)
    b1r = b1.reshape(1, H)
    b2r = b2.reshape(1, H)
    g1r = g1.reshape(1, H)
    g2r = g2.reshape(1, 1)
    full = lambda *shape: pl.BlockSpec(shape, lambda b: (0,) * len(shape))
    out = pl.pallas_call(
        _make_body(Bb, V, H, I),
        grid=grid,
        in_specs=[
            pl.BlockSpec((Bb, I), lambda b: (b, 0)),
            pl.BlockSpec((Bb * V, H), lambda b: (b, 0)),
            pl.BlockSpec(memory_space=pltpu.MemorySpace.HBM),
            full(H, 2 * H),
            full(1, H),
            full(H, H),
            full(1, H),
            full(H, H),
            full(1, H),
            full(1, H),
            full(1, 1),
        ],
        out_specs=pl.BlockSpec((Bb * V, H), lambda b: (b, 0)),
        out_shape=jax.ShapeDtypeStruct((B * V, H), jnp.float32),
        scratch_shapes=[
            pltpu.VMEM((2, I, Bb, H), jnp.float32),
            pltpu.SemaphoreType.DMA((2, I)),
        ],
        compiler_params=pltpu.CompilerParams(
            dimension_semantics=("arbitrary",)),
    )(intervention_indices, vs2, intervention_values,
      W1, b1r, W2, b2r, G1, g1r, G2, g2r)
    return out.reshape(B, V, H)


def kernel(variable_states, edge_probs, intervention_indices,
           intervention_values, W1, b1, W2, b2, G1, g1, G2, g2):
    del edge_probs
    return _run(variable_states, intervention_indices, intervention_values,
                W1, b1, W2, b2, G1, g1, G2, g2)
